# baseline (speedup 1.0000x reference)
import functools

import jax
import jax.numpy as jnp
from jax import lax
from jax.experimental import pallas as pl
from jax.experimental.pallas import tpu as pltpu
from jax.experimental.pallas import tpu_sc as plsc

D = 128
BLK = 128
CH = 8960


@functools.partial(jax.jit, static_argnums=(2,))
def _sc_lookup(idx_flat, table, n_idx):
    v_rows = table.shape[0]
    info = plsc.get_sparse_core_info()
    nc, ns = info.num_cores, info.num_subcores
    nw = nc * ns
    n_per_w = n_idx // nw
    n_blk = n_per_w // BLK
    n_pass = -(-v_rows // CH)

    mesh = plsc.VectorSubcoreMesh(core_axis_name="c", subcore_axis_name="s")

    @functools.partial(
        pl.kernel,
        mesh=mesh,
        out_type=jax.ShapeDtypeStruct((n_idx, D), jnp.float32),
        scratch_types=[
            pltpu.VMEM_SHARED((CH, D), jnp.float32),
            pltpu.VMEM((n_per_w,), jnp.int32),
            pltpu.VMEM((BLK, D), jnp.float32),
            pltpu.VMEM((BLK, D), jnp.float32),
            pltpu.VMEM((BLK,), jnp.int32),
            pltpu.VMEM((BLK,), jnp.int32),
            pltpu.VMEM((BLK,), jnp.int32),
            pltpu.VMEM((BLK,), jnp.int32),
            pltpu.SemaphoreType.DMA,
            pltpu.SemaphoreType.DMA,
            pltpu.SemaphoreType.DMA,
            pltpu.SemaphoreType.DMA,
        ],
    )
    def k(idx_hbm, table_hbm, out_hbm, chunk_sh, idx_v,
          gbuf0, gbuf1, gidx0, gidx1, spos0, spos1,
          gs0, gs1, ss0, ss1):
        sid = lax.axis_index("s")
        wid = sid * nc + lax.axis_index("c")
        base = wid * n_per_w
        gbuf = (gbuf0, gbuf1)
        gidx = (gidx0, gidx1)
        spos = (spos0, spos1)
        gsem = (gs0, gs1)
        ssem = (ss0, ss1)
        lanes = lax.iota(jnp.int32, 16)

        pltpu.sync_copy(idx_hbm.at[pl.ds(base, n_per_w)], idx_v)

        def one_pass(lo, rows_p):
            stride = (rows_p // ns) & ~7
            size = rows_p - (ns - 1) * stride

            pltpu.sync_copy(
                table_hbm.at[pl.ds(lo + sid * stride, size)],
                chunk_sh.at[pl.ds(sid * stride, size)],
            )
            plsc.subcore_barrier()

            def compute(i, slot):
                for j in range(BLK // 16):
                    off = i * BLK + j * 16
                    v = idx_v[pl.ds(off, 16)]
                    m = (v >= lo) & (v < lo + rows_p)
                    gidx[slot][pl.ds(j * 16, 16)] = jnp.where(m, v - lo, -1)
                    spos[slot][pl.ds(j * 16, 16)] = jnp.where(
                        m, base + off + lanes, -1
                    )

            def fire_gather(slot):
                pltpu.async_copy(
                    chunk_sh.at[plsc.Indices(gidx[slot], ignored_value=-1)],
                    gbuf[slot],
                    gsem[slot],
                )

            def wait_gather(slot):
                pltpu.make_async_copy(
                    chunk_sh.at[plsc.Indices(gidx[slot], ignored_value=-1)],
                    gbuf[slot],
                    gsem[slot],
                ).wait()

            def fire_scatter(slot):
                pltpu.async_copy(
                    gbuf[slot],
                    out_hbm.at[plsc.Indices(spos[slot], ignored_value=-1)],
                    ssem[slot],
                )

            def wait_scatter(slot):
                pltpu.make_async_copy(
                    gbuf[slot],
                    out_hbm.at[plsc.Indices(spos[slot], ignored_value=-1)],
                    ssem[slot],
                ).wait()

            compute(0, 0)
            fire_gather(0)

            def body(g, carry):
                for b in range(2):
                    i = 2 * g + b
                    ob = 1 - b

                    @pl.when(i >= 1)
                    def _():
                        wait_scatter(ob)

                    @pl.when(i + 1 < n_blk)
                    def _():
                        compute(i + 1, ob)
                        fire_gather(ob)

                    wait_gather(b)
                    fire_scatter(b)
                return carry

            lax.fori_loop(0, n_blk // 2, body, 0)
            wait_scatter(1)
            plsc.subcore_barrier()

        n_full = v_rows // CH
        tail = v_rows - n_full * CH

        def pass_body(p, carry):
            one_pass(p * CH, CH)
            return carry

        lax.fori_loop(0, n_full, pass_body, 0)
        if tail:
            one_pass(n_full * CH, tail)

    return k(idx_flat, table)


def kernel(nodes, ent_features):
    b, l = nodes.shape
    n = b * l
    idx_flat = nodes.reshape(n).astype(jnp.int32)
    out = _sc_lookup(idx_flat, ent_features, n)
    return out.reshape(b, l, D)


# speedup vs baseline: 5.4652x; 5.4652x over previous
import functools

import jax
import jax.numpy as jnp
from jax import lax
from jax.experimental import pallas as pl
from jax.experimental.pallas import tpu as pltpu
from jax.experimental.pallas import tpu_sc as plsc

D = 128
IDX_LANES = 128
K = 2
CHUNK = K * IDX_LANES


@functools.partial(jax.jit, static_argnums=(2, 3))
def _sc_gather(idx_rows, table, n_rows, n_idx_rows):
    info = plsc.get_sparse_core_info()
    nc, ns = info.num_cores, info.num_subcores
    nw = nc * ns
    rows_per_w = n_idx_rows // nw
    n_iter = rows_per_w // K

    mesh = plsc.VectorSubcoreMesh(core_axis_name="c", subcore_axis_name="s")

    @functools.partial(
        pl.kernel,
        mesh=mesh,
        out_type=jax.ShapeDtypeStruct((n_idx_rows * IDX_LANES, D), jnp.float32),
        scratch_types=[
            pltpu.VMEM((rows_per_w, IDX_LANES), jnp.int32),
            pltpu.VMEM((CHUNK, D), jnp.float32),
            pltpu.VMEM((CHUNK, D), jnp.float32),
            pltpu.SemaphoreType.DMA,
            pltpu.SemaphoreType.DMA,
            pltpu.SemaphoreType.DMA,
            pltpu.SemaphoreType.DMA,
        ],
    )
    def k(idx_hbm, table_hbm, out_hbm, idx_v, rows0, rows1, g0, g1, o0, o1):
        wid = lax.axis_index("s") * nc + lax.axis_index("c")
        row0 = wid * rows_per_w
        out0 = row0 * IDX_LANES
        rows = (rows0, rows1)
        gsem = (g0, g1)
        osem = (o0, o1)

        pltpu.sync_copy(idx_hbm.at[pl.ds(row0, rows_per_w)], idx_v)

        def fire(slot, it):
            for j in range(K):
                pltpu.async_copy(
                    table_hbm.at[idx_v.at[it * K + j]],
                    rows[slot].at[pl.ds(j * IDX_LANES, IDX_LANES)],
                    gsem[slot],
                )

        def drain_gather(slot):
            for j in range(K):
                pltpu.make_async_copy(
                    table_hbm.at[idx_v.at[j]],
                    rows[slot].at[pl.ds(j * IDX_LANES, IDX_LANES)],
                    gsem[slot],
                ).wait()

        def drain_out(slot):
            pltpu.make_async_copy(
                out_hbm.at[pl.ds(0, CHUNK)], rows[slot], osem[slot]
            ).wait()

        fire(0, 0)

        def body(g, carry):
            for b in range(2):
                it = 2 * g + b
                nb = 1 - b

                @pl.when(it >= 1)
                def _():
                    drain_out(nb)

                @pl.when(it + 1 < n_iter)
                def _():
                    fire(nb, it + 1)

                drain_gather(b)
                pltpu.async_copy(
                    rows[b], out_hbm.at[pl.ds(out0 + it * CHUNK, CHUNK)], osem[b]
                )

            return carry

        lax.fori_loop(0, n_iter // 2, body, 0)
        drain_out(1)

    return k(idx_rows, table)


def kernel(nodes, ent_features):
    b, l = nodes.shape
    n = b * l
    idx_rows = nodes.reshape(n // IDX_LANES, IDX_LANES).astype(jnp.int32)
    out = _sc_gather(idx_rows, ent_features,
                     ent_features.shape[0], n // IDX_LANES)
    return out.reshape(b, l, D)
